# baseline (speedup 1.0000x reference)
import functools

import jax
import jax.numpy as jnp
from jax import lax
from jax.experimental import pallas as pl
from jax.experimental.pallas import tpu as pltpu
from jax.experimental.pallas import tpu_sc as plsc

NC = 2
NS = 16
NW = NC * NS
L = 16
CHUNK = 128
G = 8
NBUF = 2
F0 = 0.25

_mesh = functools.partial(
    plsc.VectorSubcoreMesh,
    core_axis_name="c", subcore_axis_name="s",
    num_cores=NC, num_subcores=NS,
)


def _deg_kernel(n_pad, n0, n1, interpret=False):
    @functools.partial(
        pl.kernel,
        out_type=jax.ShapeDtypeStruct((NC, n_pad), jnp.float32),
        mesh=_mesh(),
        scratch_types=[
            pltpu.VMEM_SHARED((n_pad,), jnp.float32),
            pltpu.VMEM((max(n0, n1), CHUNK), jnp.int32),
            pltpu.VMEM((CHUNK,), jnp.float32),
            pltpu.SemaphoreType.DMA,
        ],
        interpret=interpret,
    )
    def deg_kernel(dst3_hbm, zeros1_hbm, deg_out, deg_sh, didx, ones_v, ssem):
        c = lax.axis_index("c")
        s = lax.axis_index("s")
        for i in range(CHUNK // L):
            ones_v[pl.ds(i * L, L)] = jnp.full((L,), 1.0, jnp.float32)

        @pl.when(s == 0)
        def _():
            pltpu.sync_copy(zeros1_hbm, deg_sh)

        tile = c * NS + s
        nch = lax.select(c == 0, n0, n1)
        pltpu.sync_copy(dst3_hbm.at[tile], didx)
        plsc.subcore_barrier()

        def body(j, carry):
            pltpu.async_copy(ones_v, deg_sh.at[didx.at[j]], ssem, add=True)
            return carry

        lax.fori_loop(0, nch, body, 0)

        def drain(j, carry):
            pltpu.make_async_copy(ones_v, deg_sh.at[didx.at[0]], ssem).wait()
            return carry

        lax.fori_loop(0, nch, drain, 0)
        plsc.subcore_barrier()

        @pl.when(s == 0)
        def _():
            pltpu.sync_copy(deg_sh, deg_out.at[c])

    return deg_kernel


def _agg_kernel(n_pad, nfeat, n0, n1, interpret=False):
    rpt = n_pad // NS

    @functools.partial(
        pl.kernel,
        out_type=jax.ShapeDtypeStruct((NC, n_pad, nfeat), jnp.float32),
        mesh=_mesh(),
        scratch_types=[
            pltpu.VMEM_SHARED((n_pad, nfeat), jnp.float32),
            pltpu.VMEM((G, CHUNK), jnp.int32),
            pltpu.VMEM((G, CHUNK), jnp.int32),
            pltpu.VMEM((NBUF, CHUNK, nfeat), jnp.float32),
            pltpu.SemaphoreType.DMA,
            pltpu.SemaphoreType.DMA,
            pltpu.SemaphoreType.DMA,
            pltpu.SemaphoreType.DMA,
        ],
        interpret=interpret,
    )
    def agg_kernel(src3_hbm, dst3_hbm, g_hbm, zeros2_hbm, out_hbm,
                   acc_sh, sidx, didx, rows, sem0, sem1, sem2, sem3):
        c = lax.axis_index("c")
        s = lax.axis_index("s")
        sems = (sem0, sem1, sem2, sem3)

        @pl.when(c == 0)
        def _():
            pltpu.sync_copy(g_hbm.at[pl.ds(s * rpt, rpt)],
                            acc_sh.at[pl.ds(s * rpt, rpt)])

        @pl.when(c == 1)
        def _():
            pltpu.sync_copy(zeros2_hbm.at[pl.ds(s * rpt, rpt)],
                            acc_sh.at[pl.ds(s * rpt, rpt)])

        tile = c * NS + s
        plsc.subcore_barrier()

        def gather(jj, b):
            pltpu.async_copy(g_hbm.at[sidx.at[jj]], rows.at[b], sems[b])

        def group(k, carry):
            pltpu.sync_copy(src3_hbm.at[tile, pl.ds(k * G, G)], sidx)
            pltpu.sync_copy(dst3_hbm.at[tile, pl.ds(k * G, G)], didx)
            for b in range(NBUF - 1):
                gather(b, b)
            for jj in range(G):
                b = jj % NBUF
                if jj + NBUF - 1 < G:
                    gather(jj + NBUF - 1, (jj + NBUF - 1) % NBUF)
                pltpu.make_async_copy(g_hbm.at[sidx.at[jj]], rows.at[b],
                                      sems[b]).wait()
                pltpu.sync_copy(rows.at[b], acc_sh.at[didx.at[jj]], add=True)
            return carry

        ngrp = lax.select(c == 0, n0 // G, n1 // G)
        lax.fori_loop(0, ngrp, group, 0)
        plsc.subcore_barrier()
        pltpu.sync_copy(acc_sh.at[pl.ds(s * rpt, rpt)],
                        out_hbm.at[c, pl.ds(s * rpt, rpt)])

    return agg_kernel


def _dense1(xp, w, deg3, interpret=False):
    n_pad, nfeat = xp.shape
    nhid = w.shape[1]
    nrow = n_pad // 128

    def body(x_ref, w_ref, deg_ref, g_ref, d_ref):
        h = jnp.dot(x_ref[...], w_ref[...], preferred_element_type=jnp.float32)
        d2 = lax.rsqrt(deg_ref[0] + deg_ref[1] + 1.0)
        d_ref[...] = d2
        g3 = h.reshape(nrow, 128, nhid) * d2[:, :, None]
        g_ref[...] = g3.reshape(n_pad, nhid)

    return pl.pallas_call(
        body,
        out_shape=(
            jax.ShapeDtypeStruct((n_pad, nhid), jnp.float32),
            jax.ShapeDtypeStruct((nrow, 128), jnp.float32),
        ),
        interpret=interpret,
    )(xp, w, deg3)


def _dense2(acc3, d2, b, interpret=False):
    n_pad, nhid = acc3.shape[1], acc3.shape[2]
    nrow = n_pad // 128

    def body(acc_ref, d_ref, b_ref, o_ref):
        t = (acc_ref[0] + acc_ref[1]).reshape(nrow, 128, nhid)
        o = t * d_ref[...][:, :, None] + b_ref[...]
        o_ref[...] = o.reshape(n_pad, nhid)

    return pl.pallas_call(
        body,
        out_shape=jax.ShapeDtypeStruct((n_pad, nhid), jnp.float32),
        interpret=interpret,
    )(acc3, d2, b)


def _gcn(x, edge_index, w, b, interpret=False):
    n, nfeat = x.shape
    nhid = w.shape[1]
    e = edge_index.shape[1]

    n_pad = ((n + 1 + 127) // 128) * 128
    total = -(-e // (NS * CHUNK))
    total = -(-total // (2 * G)) * (2 * G)
    n0 = max(G, int(round(total * F0 / G)) * G)
    n1 = total - n0
    nmax = max(n0, n1)
    ep = NS * CHUNK * total

    sflat = jnp.concatenate(
        [edge_index[0], jnp.zeros((ep - e,), edge_index.dtype)])
    pad_dst = (n + jnp.arange(ep - e, dtype=edge_index.dtype)
               % jnp.int32(n_pad - n))
    dflat = jnp.concatenate([edge_index[1], pad_dst])

    def part(flat):
        a = flat[:NS * n0 * CHUNK].reshape(NS, n0, CHUNK)
        bb = flat[NS * n0 * CHUNK:].reshape(NS, n1, CHUNK)
        a = jnp.pad(a, ((0, 0), (0, nmax - n0), (0, 0)))
        bb = jnp.pad(bb, ((0, 0), (0, nmax - n1), (0, 0)))
        return jnp.concatenate([a, bb], axis=0)

    src3 = part(sflat)
    dst3 = part(dflat)
    xp = jnp.pad(x, ((0, n_pad - n), (0, 0)))
    zeros1 = jnp.zeros((n_pad,), jnp.float32)
    zeros2 = jnp.zeros((n_pad, nhid), jnp.float32)

    deg2 = _deg_kernel(n_pad, n0, n1, interpret)(dst3, zeros1)
    g_pad, d2 = _dense1(xp, w, deg2.reshape(NC, n_pad // 128, 128), interpret)
    acc2 = _agg_kernel(n_pad, nhid, n0, n1, interpret)(src3, dst3, g_pad,
                                                       zeros2)
    out_pad = _dense2(acc2, d2, b, interpret)
    return out_pad[:n]


def kernel(x, edge_index, W, b):
    return _gcn(x, edge_index, W, b)


# speedup vs baseline: 1.1448x; 1.1448x over previous
import functools

import jax
import jax.numpy as jnp
from jax import lax
from jax.experimental import pallas as pl
from jax.experimental.pallas import tpu as pltpu
from jax.experimental.pallas import tpu_sc as plsc

NC = 2
NS = 16
NW = NC * NS
L = 16
CHUNK = 128
G = 8
NBUF = 2
F0 = 0.5

_mesh = functools.partial(
    plsc.VectorSubcoreMesh,
    core_axis_name="c", subcore_axis_name="s",
    num_cores=NC, num_subcores=NS,
)


def _deg_kernel(n_pad, n0, n1, interpret=False):
    @functools.partial(
        pl.kernel,
        out_type=jax.ShapeDtypeStruct((NC, n_pad), jnp.float32),
        mesh=_mesh(),
        scratch_types=[
            pltpu.VMEM_SHARED((n_pad,), jnp.float32),
            pltpu.VMEM((max(n0, n1), CHUNK), jnp.int32),
            pltpu.VMEM((CHUNK,), jnp.float32),
            pltpu.SemaphoreType.DMA,
        ],
        interpret=interpret,
    )
    def deg_kernel(dst3_hbm, zeros1_hbm, deg_out, deg_sh, didx, ones_v, ssem):
        c = lax.axis_index("c")
        s = lax.axis_index("s")
        for i in range(CHUNK // L):
            ones_v[pl.ds(i * L, L)] = jnp.full((L,), 1.0, jnp.float32)

        @pl.when(s == 0)
        def _():
            pltpu.sync_copy(zeros1_hbm, deg_sh)

        tile = c * NS + s
        nch = lax.select(c == 0, n0, n1)
        pltpu.sync_copy(dst3_hbm.at[tile], didx)
        plsc.subcore_barrier()

        def body(j, carry):
            pltpu.async_copy(ones_v, deg_sh.at[didx.at[j]], ssem, add=True)
            return carry

        lax.fori_loop(0, nch, body, 0)

        def drain(j, carry):
            pltpu.make_async_copy(ones_v, deg_sh.at[didx.at[0]], ssem).wait()
            return carry

        lax.fori_loop(0, nch, drain, 0)
        plsc.subcore_barrier()

        @pl.when(s == 0)
        def _():
            pltpu.sync_copy(deg_sh, deg_out.at[c])

    return deg_kernel


def _agg_kernel(n_pad, nfeat, n0, n1, interpret=False):
    rpt = n_pad // NS

    @functools.partial(
        pl.kernel,
        out_type=jax.ShapeDtypeStruct((NC, n_pad, nfeat), jnp.float32),
        mesh=_mesh(),
        scratch_types=[
            pltpu.VMEM_SHARED((n_pad, nfeat), jnp.float32),
            pltpu.VMEM((G, CHUNK), jnp.int32),
            pltpu.VMEM((G, CHUNK), jnp.int32),
            pltpu.VMEM((NBUF, CHUNK, nfeat), jnp.float32),
            pltpu.SemaphoreType.DMA,
            pltpu.SemaphoreType.DMA,
            pltpu.SemaphoreType.DMA,
            pltpu.SemaphoreType.DMA,
        ],
        interpret=interpret,
    )
    def agg_kernel(src3_hbm, dst3_hbm, g_hbm, zeros2_hbm, out_hbm,
                   acc_sh, sidx, didx, rows, sem0, sem1, sem2, sem3):
        c = lax.axis_index("c")
        s = lax.axis_index("s")
        sems = (sem0, sem1, sem2, sem3)

        @pl.when(c == 0)
        def _():
            pltpu.sync_copy(g_hbm.at[pl.ds(s * rpt, rpt)],
                            acc_sh.at[pl.ds(s * rpt, rpt)])

        @pl.when(c == 1)
        def _():
            pltpu.sync_copy(zeros2_hbm.at[pl.ds(s * rpt, rpt)],
                            acc_sh.at[pl.ds(s * rpt, rpt)])

        tile = c * NS + s
        plsc.subcore_barrier()

        def gather(jj, b):
            pltpu.async_copy(g_hbm.at[sidx.at[jj]], rows.at[b], sems[b])

        def group(k, carry):
            pltpu.sync_copy(src3_hbm.at[tile, pl.ds(k * G, G)], sidx)
            pltpu.sync_copy(dst3_hbm.at[tile, pl.ds(k * G, G)], didx)
            for b in range(NBUF - 1):
                gather(b, b)
            for jj in range(G):
                b = jj % NBUF
                if jj + NBUF - 1 < G:
                    gather(jj + NBUF - 1, (jj + NBUF - 1) % NBUF)
                pltpu.make_async_copy(g_hbm.at[sidx.at[jj]], rows.at[b],
                                      sems[b]).wait()
                pltpu.sync_copy(rows.at[b], acc_sh.at[didx.at[jj]], add=True)
            return carry

        ngrp = lax.select(c == 0, n0 // G, n1 // G)
        lax.fori_loop(0, ngrp, group, 0)
        plsc.subcore_barrier()
        pltpu.sync_copy(acc_sh.at[pl.ds(s * rpt, rpt)],
                        out_hbm.at[c, pl.ds(s * rpt, rpt)])

    return agg_kernel


def _dense1(xp, w, deg3, interpret=False):
    n_pad, nfeat = xp.shape
    nhid = w.shape[1]
    nrow = n_pad // 128

    def body(x_ref, w_ref, deg_ref, g_ref, d_ref):
        h = jnp.dot(x_ref[...], w_ref[...], preferred_element_type=jnp.float32)
        d2 = lax.rsqrt(deg_ref[0] + deg_ref[1] + 1.0)
        d_ref[...] = d2
        g3 = h.reshape(nrow, 128, nhid) * d2[:, :, None]
        g_ref[...] = g3.reshape(n_pad, nhid)

    return pl.pallas_call(
        body,
        out_shape=(
            jax.ShapeDtypeStruct((n_pad, nhid), jnp.float32),
            jax.ShapeDtypeStruct((nrow, 128), jnp.float32),
        ),
        interpret=interpret,
    )(xp, w, deg3)


def _dense2(acc3, d2, b, interpret=False):
    n_pad, nhid = acc3.shape[1], acc3.shape[2]
    nrow = n_pad // 128

    def body(acc_ref, d_ref, b_ref, o_ref):
        t = (acc_ref[0] + acc_ref[1]).reshape(nrow, 128, nhid)
        o = t * d_ref[...][:, :, None] + b_ref[...]
        o_ref[...] = o.reshape(n_pad, nhid)

    return pl.pallas_call(
        body,
        out_shape=jax.ShapeDtypeStruct((n_pad, nhid), jnp.float32),
        interpret=interpret,
    )(acc3, d2, b)


def _gcn(x, edge_index, w, b, interpret=False):
    n, nfeat = x.shape
    nhid = w.shape[1]
    e = edge_index.shape[1]

    n_pad = ((n + 1 + 127) // 128) * 128
    total = -(-e // (NS * CHUNK))
    total = -(-total // (2 * G)) * (2 * G)
    n0 = max(G, int(round(total * F0 / G)) * G)
    n1 = total - n0
    nmax = max(n0, n1)
    ep = NS * CHUNK * total

    sflat = jnp.concatenate(
        [edge_index[0], jnp.zeros((ep - e,), edge_index.dtype)])
    pad_dst = (n + jnp.arange(ep - e, dtype=edge_index.dtype)
               % jnp.int32(n_pad - n))
    dflat = jnp.concatenate([edge_index[1], pad_dst])

    def part(flat):
        a = flat[:NS * n0 * CHUNK].reshape(NS, n0, CHUNK)
        bb = flat[NS * n0 * CHUNK:].reshape(NS, n1, CHUNK)
        a = jnp.pad(a, ((0, 0), (0, nmax - n0), (0, 0)))
        bb = jnp.pad(bb, ((0, 0), (0, nmax - n1), (0, 0)))
        return jnp.concatenate([a, bb], axis=0)

    src3 = part(sflat)
    dst3 = part(dflat)
    xp = jnp.pad(x, ((0, n_pad - n), (0, 0)))
    zeros1 = jnp.zeros((n_pad,), jnp.float32)
    zeros2 = jnp.zeros((n_pad, nhid), jnp.float32)

    deg2 = _deg_kernel(n_pad, n0, n1, interpret)(dst3, zeros1)
    g_pad, d2 = _dense1(xp, w, deg2.reshape(NC, n_pad // 128, 128), interpret)
    acc2 = _agg_kernel(n_pad, nhid, n0, n1, interpret)(src3, dst3, g_pad,
                                                       zeros2)
    out_pad = _dense2(acc2, d2, b, interpret)
    return out_pad[:n]


def kernel(x, edge_index, W, b):
    return _gcn(x, edge_index, W, b)
